# baseline (speedup 1.0000x reference)
import functools

import jax
import jax.numpy as jnp
from jax import lax
from jax.experimental import pallas as pl
from jax.experimental.pallas import tpu as pltpu
from jax.experimental.pallas import tpu_sc as plsc

_NC = 2
_NS = 16
_NW = _NC * _NS


_CHUNK = 128


def _sc_gather_flat(flat, idx_e):
    M = idx_e.shape[0]
    m_per_w = M // _NW
    n_chunks = m_per_w // _CHUNK
    mesh = plsc.VectorSubcoreMesh(core_axis_name="c", subcore_axis_name="s")

    @functools.partial(
        pl.kernel,
        mesh=mesh,
        out_type=jax.ShapeDtypeStruct((M,), jnp.float32),
        scratch_types=[
            pltpu.VMEM((m_per_w,), jnp.int32),
            pltpu.VMEM((m_per_w,), jnp.float32),
            pltpu.SemaphoreType.DMA,
        ],
    )
    def gather_kernel(flat_hbm, idx_hbm, out_hbm, idx_v, vals_v, sem):
        wid = lax.axis_index("s") * _NC + lax.axis_index("c")
        base = wid * m_per_w
        pltpu.sync_copy(idx_hbm.at[pl.ds(base, m_per_w)], idx_v)

        @pl.loop(0, n_chunks)
        def _(c):
            off = c * _CHUNK
            pltpu.async_copy(
                flat_hbm.at[idx_v.at[pl.ds(off, _CHUNK)]],
                vals_v.at[pl.ds(off, _CHUNK)],
                sem,
            ).wait()

        pltpu.sync_copy(vals_v, out_hbm.at[pl.ds(base, m_per_w)])

    return gather_kernel(flat, idx_e)


def _mlp_kernel(x_ref, w1_ref, b1_ref, w2_ref, b2_ref, w3_ref, b3_ref,
                wo_ref, bo_ref, o_ref):
    h = jnp.dot(x_ref[...], w1_ref[...], preferred_element_type=jnp.float32)
    h = jnp.maximum(h + b1_ref[...], 0.0)
    h = jnp.dot(h, w2_ref[...], preferred_element_type=jnp.float32)
    h = jnp.maximum(h + b2_ref[...], 0.0)
    h = jnp.dot(h, w3_ref[...], preferred_element_type=jnp.float32)
    h = jnp.maximum(h + b3_ref[...], 0.0)
    o_ref[...] = (
        jnp.dot(h, wo_ref[...], preferred_element_type=jnp.float32)
        + bo_ref[...]
    )


def _tc_mlp(embeds, W1T, b1, W2T, b2, W3T, b3, WoT, bo, blk):
    B, D = embeds.shape
    H = W2T.shape[0]
    O = WoT.shape[1]
    full = lambda shape: pl.BlockSpec(shape, lambda i: (0, 0))
    return pl.pallas_call(
        _mlp_kernel,
        grid=(B // blk,),
        in_specs=[
            pl.BlockSpec((blk, D), lambda i: (i, 0)),
            full((D, H)), full((1, H)),
            full((H, H)), full((1, H)),
            full((H, H)), full((1, H)),
            full((H, O)), full((1, O)),
        ],
        out_specs=pl.BlockSpec((blk, O), lambda i: (i, 0)),
        out_shape=jax.ShapeDtypeStruct((B, O), jnp.float32),
    )(embeds, W1T, b1, W2T, b2, W3T, b3, WoT, bo)


def kernel(x, emb, W1, b1, W2, b2, W3, b3, Wm, bm, Wc, bc, Ws, bs):
    B = x.shape[0]
    D = emb.shape[1]
    idx = x[:, 0].astype(jnp.int32)
    idx_e = (idx[:, None] * D + jnp.arange(D, dtype=jnp.int32)).reshape(-1)
    embeds = _sc_gather_flat(emb.reshape(-1), idx_e).reshape(B, D)
    WoT = jnp.concatenate([Wm, Wc, Ws], axis=0).T
    bo = jnp.concatenate([bm, bc, bs], axis=0)[None, :]
    return _tc_mlp(embeds, W1.T, b1[None, :], W2.T, b2[None, :],
                   W3.T, b3[None, :], WoT, bo, blk=2048)


# speedup vs baseline: 1.9970x; 1.9970x over previous
import functools

import jax
import jax.numpy as jnp
from jax import lax
from jax.experimental import pallas as pl
from jax.experimental.pallas import tpu as pltpu
from jax.experimental.pallas import tpu_sc as plsc

_NC = 2
_NS = 16
_NW = _NC * _NS


def _sc_gather_rows(emb, idx):
    B = idx.shape[0]
    D = emb.shape[1]
    b_per_w = B // _NW
    mesh = plsc.VectorSubcoreMesh(core_axis_name="c", subcore_axis_name="s")

    @functools.partial(
        pl.kernel,
        mesh=mesh,
        out_type=jax.ShapeDtypeStruct((B, D), jnp.float32),
        scratch_types=[
            pltpu.VMEM((b_per_w,), jnp.int32),
            pltpu.VMEM((b_per_w, D), jnp.float32),
            pltpu.SemaphoreType.DMA,
        ],
    )
    def gather_kernel(table_hbm, idx_hbm, out_hbm, idx_v, rows_v, sem):
        wid = lax.axis_index("s") * _NC + lax.axis_index("c")
        base = wid * b_per_w
        pltpu.sync_copy(idx_hbm.at[pl.ds(base, b_per_w)], idx_v)

        @pl.loop(0, b_per_w // 16)
        def _(g):
            vec = idx_v[pl.ds(g * 16, 16)]
            for j in range(16):
                pltpu.make_async_copy(
                    table_hbm.at[pl.ds(vec[j], 1)],
                    rows_v.at[pl.ds(g * 16 + j, 1)],
                    sem,
                ).start()

        @pl.loop(0, b_per_w)
        def _(k):
            pltpu.make_async_copy(
                table_hbm.at[pl.ds(0, 1)], rows_v.at[pl.ds(0, 1)], sem
            ).wait()

        pltpu.sync_copy(rows_v, out_hbm.at[pl.ds(base, b_per_w)])

    return gather_kernel(emb, idx)


def _mlp_kernel(x_ref, w1_ref, b1_ref, w2_ref, b2_ref, w3_ref, b3_ref,
                wo_ref, bo_ref, o_ref):
    h = jnp.dot(x_ref[...], w1_ref[...], preferred_element_type=jnp.float32)
    h = jnp.maximum(h + b1_ref[...], 0.0)
    h = jnp.dot(h, w2_ref[...], preferred_element_type=jnp.float32)
    h = jnp.maximum(h + b2_ref[...], 0.0)
    h = jnp.dot(h, w3_ref[...], preferred_element_type=jnp.float32)
    h = jnp.maximum(h + b3_ref[...], 0.0)
    o_ref[...] = (
        jnp.dot(h, wo_ref[...], preferred_element_type=jnp.float32)
        + bo_ref[...]
    )


def _tc_mlp(embeds, W1T, b1, W2T, b2, W3T, b3, WoT, bo, blk):
    B, D = embeds.shape
    H = W2T.shape[0]
    O = WoT.shape[1]
    full = lambda shape: pl.BlockSpec(shape, lambda i: (0, 0))
    return pl.pallas_call(
        _mlp_kernel,
        grid=(B // blk,),
        in_specs=[
            pl.BlockSpec((blk, D), lambda i: (i, 0)),
            full((D, H)), full((1, H)),
            full((H, H)), full((1, H)),
            full((H, H)), full((1, H)),
            full((H, O)), full((1, O)),
        ],
        out_specs=pl.BlockSpec((blk, O), lambda i: (i, 0)),
        out_shape=jax.ShapeDtypeStruct((B, O), jnp.float32),
    )(embeds, W1T, b1, W2T, b2, W3T, b3, WoT, bo)


def kernel(x, emb, W1, b1, W2, b2, W3, b3, Wm, bm, Wc, bc, Ws, bs):
    idx = x[:, 0].astype(jnp.int32)
    embeds = _sc_gather_rows(emb, idx)
    WoT = jnp.concatenate([Wm, Wc, Ws], axis=0).T
    bo = jnp.concatenate([bm, bc, bs], axis=0)[None, :]
    return _tc_mlp(embeds, W1.T, b1[None, :], W2.T, b2[None, :],
                   W3.T, b3[None, :], WoT, bo, blk=2048)
